# baseline (speedup 1.0000x reference)
import jax
import jax.numpy as jnp
from jax.experimental import pallas as pl
from jax.experimental.pallas import tpu as pltpu

_D = 1024
_E = 16
_F = 4096
_FH = _F // 2
_DH = _D // 2
_N = 256
_INV_SQRT2 = 0.7071067811865476


def _moe_step(x_ref, wr_ref, w1_ref, b1_ref, w2_ref, b2_ref, out_ref,
              a_ref, acc_ref):
    g = pl.program_id(0)
    x = x_ref[...]

    @pl.when(g < 2 * _E)
    def _w1_path():
        f = g % 2
        par = (g // 2) % 2
        h = jax.lax.dot_general(x, w1_ref[0], (((1,), (1,)), ((), ())),
                                preferred_element_type=jnp.float32)
        h = h + b1_ref[0]
        a = 0.5 * h * (1.0 + jax.lax.erf(h * _INV_SQRT2))
        a_ref[par, :, pl.ds(f * _FH, _FH)] = a

    @pl.when(g >= 1)
    def _w2_path():
        ev = (g - 1) // 2
        fv = (g - 1) % 2
        parv = ev % 2
        a_full = a_ref[parv]
        o_part = jax.lax.dot_general(a_full, w2_ref[0], (((1,), (1,)), ((), ())),
                                     preferred_element_type=jnp.float32)
        acc_ref[:, pl.ds(fv * _DH, _DH)] = o_part

        @pl.when(fv == 1)
        def _combine():
            logits = jax.lax.dot_general(x, wr_ref[...], (((1,), (1,)), ((), ())),
                                         preferred_element_type=jnp.float32)
            col = jax.lax.broadcasted_iota(jnp.int32, logits.shape, 1)
            m1 = jnp.max(logits, axis=-1)
            a1 = jnp.min(jnp.where(logits == m1[:, None], col, _E), axis=-1)
            neg = jnp.finfo(jnp.float32).min
            logits2 = jnp.where(col == a1[:, None], neg, logits)
            m2 = jnp.max(logits2, axis=-1)
            a2 = jnp.min(jnp.where(logits2 == m2[:, None], col, _E), axis=-1)
            p1 = 1.0 / (1.0 + jnp.exp(m2 - m1))
            w_e = (jnp.where(a1 == ev, p1, 0.0)
                   + jnp.where(a2 == ev, 1.0 - p1, 0.0))

            contrib = w_e[:, None] * (acc_ref[...] + b2_ref[0])

            @pl.when(ev == 0)
            def _init_out():
                out_ref[...] = contrib

            @pl.when(ev > 0)
            def _acc_out():
                out_ref[...] += contrib


def kernel(x, Wr, W1, b1, W2, b2):
    B, S, D = x.shape
    xf = x.reshape(_N, D)
    b1r = b1.reshape(_E, 1, _F)
    b2r = b2.reshape(_E, 1, _D)

    def w1_map(g):
        return (jnp.minimum(g // 2, _E - 1), jnp.where(g >= 2 * _E, 1, g % 2), 0)

    def b1_map(g):
        return (jnp.minimum(g // 2, _E - 1), 0,
                jnp.where(g >= 2 * _E, 1, g % 2))

    def w2_map(g):
        return (jnp.clip((g - 1) // 2, 0, _E - 1),
                jnp.where(g < 1, 0, (g - 1) % 2), 0)

    def b2_map(g):
        return (jnp.clip((g - 1) // 2, 0, _E - 1), 0, 0)

    out = pl.pallas_call(
        _moe_step,
        grid=(2 * _E + 1,),
        in_specs=[
            pl.BlockSpec((_N, D), lambda g: (0, 0)),
            pl.BlockSpec((_E, D), lambda g: (0, 0)),
            pl.BlockSpec((1, _FH, _D), w1_map),
            pl.BlockSpec((1, 1, _FH), b1_map),
            pl.BlockSpec((1, _DH, _F), w2_map),
            pl.BlockSpec((1, 1, _D), b2_map),
        ],
        out_specs=pl.BlockSpec((_N, _D), lambda g: (0, 0)),
        out_shape=jax.ShapeDtypeStruct((_N, _D), jnp.float32),
        scratch_shapes=[
            pltpu.VMEM((2, _N, _F), jnp.float32),
            pltpu.VMEM((_N, _D), jnp.float32),
        ],
        compiler_params=pltpu.CompilerParams(
            dimension_semantics=("arbitrary",),
        ),
    )(xf, Wr, W1, b1r, W2, b2r)
    return out.reshape(B, S, D)
